# baseline (speedup 1.0000x reference)
import functools

import jax
import jax.numpy as jnp
from jax import lax
from jax.experimental import pallas as pl
from jax.experimental.pallas import tpu as pltpu
from jax.experimental.pallas import tpu_sc as plsc

K = 512
D = 256
T = 256
COMMITMENT = 0.25

N_STAGES = 8
T_STAGE = T // N_STAGES
T_BLK = 8
N_STEPS = T_STAGE // T_BLK

_LOSS_SCALE = (1.0 + COMMITMENT) / (T * D * D)


def _tc_prologue_body(x_ref, emb_ref, f_ref, loss_ref):
    emb = emb_ref[...]
    x_full = x_ref[...]
    g = lax.dot_general(emb, x_full, (((1,), (1,)), ((), ())),
                        preferred_element_type=jnp.float32)
    enorm2 = jnp.sum(emb * emb, axis=1, keepdims=True)
    f_ref[...] = enorm2 - 2.0 * g
    loss_ref[0, 0] = _LOSS_SCALE * float(T) * jnp.sum(x_full * x_full)


def _tc_prologue(x2d, emb):
    return pl.pallas_call(
        _tc_prologue_body,
        in_specs=[
            pl.BlockSpec((T, D), lambda: (0, 0)),
            pl.BlockSpec((K, D), lambda: (0, 0)),
        ],
        out_specs=[
            pl.BlockSpec((K, T), lambda: (0, 0)),
            pl.BlockSpec(memory_space=pltpu.SMEM, block_shape=(1, 1),
                         index_map=lambda: (0, 0)),
        ],
        out_shape=[
            jax.ShapeDtypeStruct((K, T), jnp.float32),
            jax.ShapeDtypeStruct((1, 1), jnp.float32),
        ],
    )(x2d, emb)


def _tc_stage_body(loss_in_ref, x_blk_ref, f_ref, emb_ref,
                   idx_ref, loss_ref, acc_ref):
    s = pl.program_id(0)
    emb = emb_ref[...]

    @pl.when(s == 0)
    def _init():
        acc_ref[0, 0] = loss_in_ref[0, 0]

    xblk = x_blk_ref[...]
    diff = xblk[:, None, :] - emb[None, :, :]
    d = diff * diff
    m = jnp.min(d, axis=1)
    iota_k = lax.broadcasted_iota(jnp.int32, (T_BLK, K, D), 1)
    hit = d == m[:, None, :]
    idx = jnp.min(jnp.where(hit, iota_k, K), axis=1)
    idx_ref[...] = idx.reshape(T_BLK * D)

    onehot = iota_k == idx[:, None, :]
    f = f_ref[...]
    contrib = jnp.sum(jnp.where(onehot, f[None, :, :], 0.0))
    acc_ref[0, 0] += _LOSS_SCALE * contrib

    @pl.when(s == N_STEPS - 1)
    def _fin():
        loss_ref[0, 0] = acc_ref[0, 0]


def _tc_stage_call(stage, x2d, f, emb, loss_in):
    return pl.pallas_call(
        _tc_stage_body,
        grid=(N_STEPS,),
        in_specs=[
            pl.BlockSpec(memory_space=pltpu.SMEM, block_shape=(1, 1),
                         index_map=lambda s: (0, 0)),
            pl.BlockSpec((T_BLK, D), lambda s: (s + stage * N_STEPS, 0)),
            pl.BlockSpec((K, T), lambda s: (0, 0)),
            pl.BlockSpec((K, D), lambda s: (0, 0)),
        ],
        out_specs=[
            pl.BlockSpec((T_BLK * D,), lambda s: (s,)),
            pl.BlockSpec(memory_space=pltpu.SMEM, block_shape=(1, 1),
                         index_map=lambda s: (0, 0)),
        ],
        out_shape=[
            jax.ShapeDtypeStruct((T_STAGE * D,), jnp.int32),
            jax.ShapeDtypeStruct((1, 1), jnp.float32),
        ],
        scratch_shapes=[
            pltpu.SMEM((1, 1), jnp.float32),
        ],
        compiler_params=pltpu.CompilerParams(
            dimension_semantics=("arbitrary",),
        ),
    )(loss_in, x2d, f, emb)


_NC = 2
_NS = 16
_NW = _NC * _NS
_B = T * D
_B_STAGE = T_STAGE * D
_ROWS_PER_W = _B_STAGE // _NW
_CHUNK = 128
_N_CHUNKS = _ROWS_PER_W // _CHUNK


def _sc_stage_body(stage, emb_hbm, idx_hbm, out_hbm,
                   idx_all, rows_v0, rows_v1, sem0, sem1):
    wid = lax.axis_index("s") * _NC + lax.axis_index("c")
    w_base = wid * _ROWS_PER_W
    out_base = stage * _B_STAGE + w_base

    pltpu.sync_copy(idx_hbm.at[pl.ds(w_base, _ROWS_PER_W)], idx_all)

    def fire(rows_v, sem, c):
        idx_c = idx_all.at[pl.ds(c * _CHUNK, _CHUNK)]
        pltpu.make_async_copy(emb_hbm.at[idx_c], rows_v, sem).start()

    def drain(rows_v, sem, c):
        idx_c = idx_all.at[pl.ds(c * _CHUNK, _CHUNK)]
        pltpu.make_async_copy(emb_hbm.at[idx_c], rows_v, sem).wait()
        pltpu.sync_copy(rows_v, out_hbm.at[pl.ds(out_base + c * _CHUNK,
                                                 _CHUNK)])

    fire(rows_v0, sem0, 0)
    for c in range(_N_CHUNKS):
        if c + 1 < _N_CHUNKS:
            fire((rows_v1, rows_v0)[c % 2], (sem1, sem0)[c % 2], c + 1)
        drain((rows_v0, rows_v1)[c % 2], (sem0, sem1)[c % 2], c)


@functools.cache
def _sc_stage_kernel(stage):
    return pl.kernel(
        functools.partial(_sc_stage_body, stage),
        out_type=(),
        mesh=plsc.VectorSubcoreMesh(core_axis_name="c", subcore_axis_name="s",
                                    num_cores=_NC, num_subcores=_NS),
        scratch_types=[
            pltpu.VMEM((_ROWS_PER_W,), jnp.int32),
            pltpu.VMEM((_CHUNK, D), jnp.float32),
            pltpu.VMEM((_CHUNK, D), jnp.float32),
            pltpu.SemaphoreType.DMA,
            pltpu.SemaphoreType.DMA,
        ],
        name=f"sc_gather_stage{stage}",
    )


def kernel(x, embeddings):
    x2d = x[0]
    out_ref = jax.new_ref(pl.empty((_B, D), jnp.float32))
    f, loss = _tc_prologue(x2d, embeddings)
    for s in range(N_STAGES):
        idx_s, loss = _tc_stage_call(s, x2d, f, embeddings, loss)
        _sc_stage_kernel(s)(embeddings, idx_s, out_ref)
    return out_ref[...].reshape(1, T, D, D), loss[0, 0]


# speedup vs baseline: 1.0092x; 1.0092x over previous
import functools

import jax
import jax.numpy as jnp
from jax import lax
from jax.experimental import pallas as pl
from jax.experimental.pallas import tpu as pltpu
from jax.experimental.pallas import tpu_sc as plsc

K = 512
D = 256
T = 256
COMMITMENT = 0.25

N_STAGES = 4
T_STAGE = T // N_STAGES
T_BLK = 8
N_STEPS = T_STAGE // T_BLK

_LOSS_SCALE = (1.0 + COMMITMENT) / (T * D * D)


def _tc_prologue_body(x_ref, emb_ref, f_ref, loss_ref):
    emb = emb_ref[...]
    x_full = x_ref[...]
    g = lax.dot_general(emb, x_full, (((1,), (1,)), ((), ())),
                        preferred_element_type=jnp.float32)
    enorm2 = jnp.sum(emb * emb, axis=1, keepdims=True)
    f_ref[...] = enorm2 - 2.0 * g
    loss_ref[0, 0] = _LOSS_SCALE * float(T) * jnp.sum(x_full * x_full)


def _tc_prologue(x2d, emb):
    return pl.pallas_call(
        _tc_prologue_body,
        in_specs=[
            pl.BlockSpec((T, D), lambda: (0, 0)),
            pl.BlockSpec((K, D), lambda: (0, 0)),
        ],
        out_specs=[
            pl.BlockSpec((K, T), lambda: (0, 0)),
            pl.BlockSpec(memory_space=pltpu.SMEM, block_shape=(1, 1),
                         index_map=lambda: (0, 0)),
        ],
        out_shape=[
            jax.ShapeDtypeStruct((K, T), jnp.float32),
            jax.ShapeDtypeStruct((1, 1), jnp.float32),
        ],
    )(x2d, emb)


def _tc_stage_body(loss_in_ref, x_blk_ref, f_ref, emb_ref,
                   idx_ref, loss_ref, acc_ref):
    s = pl.program_id(0)
    emb = emb_ref[...]

    @pl.when(s == 0)
    def _init():
        acc_ref[0, 0] = loss_in_ref[0, 0]

    xblk = x_blk_ref[...]
    diff = xblk[:, None, :] - emb[None, :, :]
    d = diff * diff
    m = jnp.min(d, axis=1)
    iota_k = lax.broadcasted_iota(jnp.int32, (T_BLK, K, D), 1)
    hit = d == m[:, None, :]
    idx = jnp.min(jnp.where(hit, iota_k, K), axis=1)
    idx_ref[...] = idx.reshape(T_BLK * D)

    onehot = iota_k == idx[:, None, :]
    f = f_ref[...]
    contrib = jnp.sum(jnp.where(onehot, f[None, :, :], 0.0))
    acc_ref[0, 0] += _LOSS_SCALE * contrib

    @pl.when(s == N_STEPS - 1)
    def _fin():
        loss_ref[0, 0] = acc_ref[0, 0]


def _tc_stage_call(stage, x2d, f, emb, loss_in):
    return pl.pallas_call(
        _tc_stage_body,
        grid=(N_STEPS,),
        in_specs=[
            pl.BlockSpec(memory_space=pltpu.SMEM, block_shape=(1, 1),
                         index_map=lambda s: (0, 0)),
            pl.BlockSpec((T_BLK, D), lambda s: (s + stage * N_STEPS, 0)),
            pl.BlockSpec((K, T), lambda s: (0, 0)),
            pl.BlockSpec((K, D), lambda s: (0, 0)),
        ],
        out_specs=[
            pl.BlockSpec((T_BLK * D,), lambda s: (s,)),
            pl.BlockSpec(memory_space=pltpu.SMEM, block_shape=(1, 1),
                         index_map=lambda s: (0, 0)),
        ],
        out_shape=[
            jax.ShapeDtypeStruct((T_STAGE * D,), jnp.int32),
            jax.ShapeDtypeStruct((1, 1), jnp.float32),
        ],
        scratch_shapes=[
            pltpu.SMEM((1, 1), jnp.float32),
        ],
        compiler_params=pltpu.CompilerParams(
            dimension_semantics=("arbitrary",),
        ),
    )(loss_in, x2d, f, emb)


_NC = 2
_NS = 16
_NW = _NC * _NS
_B = T * D
_B_STAGE = T_STAGE * D
_ROWS_PER_W = _B_STAGE // _NW
_CHUNK = 128
_N_CHUNKS = _ROWS_PER_W // _CHUNK


def _sc_stage_body(stage, emb_hbm, idx_hbm, out_hbm,
                   idx_all, rows_v0, rows_v1, sem0, sem1):
    wid = lax.axis_index("s") * _NC + lax.axis_index("c")
    w_base = wid * _ROWS_PER_W
    out_base = stage * _B_STAGE + w_base

    pltpu.sync_copy(idx_hbm.at[pl.ds(w_base, _ROWS_PER_W)], idx_all)

    def fire(rows_v, sem, c):
        idx_c = idx_all.at[pl.ds(c * _CHUNK, _CHUNK)]
        pltpu.make_async_copy(emb_hbm.at[idx_c], rows_v, sem).start()

    def drain(rows_v, sem, c):
        idx_c = idx_all.at[pl.ds(c * _CHUNK, _CHUNK)]
        pltpu.make_async_copy(emb_hbm.at[idx_c], rows_v, sem).wait()
        pltpu.sync_copy(rows_v, out_hbm.at[pl.ds(out_base + c * _CHUNK,
                                                 _CHUNK)])

    fire(rows_v0, sem0, 0)
    for c in range(_N_CHUNKS):
        if c + 1 < _N_CHUNKS:
            fire((rows_v1, rows_v0)[c % 2], (sem1, sem0)[c % 2], c + 1)
        drain((rows_v0, rows_v1)[c % 2], (sem0, sem1)[c % 2], c)


@functools.cache
def _sc_stage_kernel(stage):
    return pl.kernel(
        functools.partial(_sc_stage_body, stage),
        out_type=(),
        mesh=plsc.VectorSubcoreMesh(core_axis_name="c", subcore_axis_name="s",
                                    num_cores=_NC, num_subcores=_NS),
        scratch_types=[
            pltpu.VMEM((_ROWS_PER_W,), jnp.int32),
            pltpu.VMEM((_CHUNK, D), jnp.float32),
            pltpu.VMEM((_CHUNK, D), jnp.float32),
            pltpu.SemaphoreType.DMA,
            pltpu.SemaphoreType.DMA,
        ],
        name=f"sc_gather_stage{stage}",
    )


def kernel(x, embeddings):
    x2d = x[0]
    out_ref = jax.new_ref(pl.empty((_B, D), jnp.float32))
    f, loss = _tc_prologue(x2d, embeddings)
    for s in range(N_STAGES):
        idx_s, loss = _tc_stage_call(s, x2d, f, embeddings, loss)
        _sc_stage_kernel(s)(embeddings, idx_s, out_ref)
    return out_ref[...].reshape(1, T, D, D), loss[0, 0]


# speedup vs baseline: 1.0686x; 1.0588x over previous
import functools

import jax
import jax.numpy as jnp
from jax import lax
from jax.experimental import pallas as pl
from jax.experimental.pallas import tpu as pltpu
from jax.experimental.pallas import tpu_sc as plsc

K = 512
D = 256
T = 256
COMMITMENT = 0.25

N_STAGES = 4
T_STAGE = T // N_STAGES
T_BLK = 8
N_STEPS = T_STAGE // T_BLK
KC = 128

_LOSS_SCALE = (1.0 + COMMITMENT) / (T * D * D)


def _tc_stage_body(loss_in_ref, x_blk_ref, x_full_ref, emb_ref,
                   idx_ref, loss_ref, f_ref, acc_ref):
    s = pl.program_id(0)
    emb = emb_ref[...]

    @pl.when(s == 0)
    def _init():
        x_full = x_full_ref[...]
        g = lax.dot_general(emb, x_full, (((1,), (1,)), ((), ())),
                            preferred_element_type=jnp.float32)
        enorm2 = jnp.sum(emb * emb, axis=1, keepdims=True)
        f_ref[...] = enorm2 - 2.0 * g
        acc_ref[0, 0] = (loss_in_ref[0, 0]
                         + _LOSS_SCALE * float(T_STAGE)
                         * jnp.sum(x_full * x_full))

    xblk = x_blk_ref[...]

    f = f_ref[...]
    m = None
    idx = None
    lvals = None
    for c in range(K // KC):
        e_c = emb[c * KC:(c + 1) * KC, :]
        f_c = f[c * KC:(c + 1) * KC, :]
        diff = xblk[:, None, :] - e_c[None, :, :]
        d = diff * diff
        mc = jnp.min(d, axis=1)
        iota_c = lax.broadcasted_iota(jnp.int32, (T_BLK, KC, D), 1) + c * KC
        hit_c = d == mc[:, None, :]
        idxc = jnp.min(jnp.where(hit_c, iota_c, K), axis=1)
        onehot_c = iota_c == idxc[:, None, :]
        lc = jnp.sum(jnp.where(onehot_c, f_c[None, :, :], 0.0), axis=1)
        if m is None:
            m, idx, lvals = mc, idxc, lc
        else:
            upd = mc < m
            idx = jnp.where(upd, idxc, idx)
            lvals = jnp.where(upd, lc, lvals)
            m = jnp.where(upd, mc, m)
    idx_ref[...] = idx.reshape(T_BLK * D)
    acc_ref[0, 0] += _LOSS_SCALE * jnp.sum(lvals)

    @pl.when(s == N_STEPS - 1)
    def _fin():
        loss_ref[0, 0] = acc_ref[0, 0]


def _tc_stage_call(stage, x2d, emb, loss_in):
    return pl.pallas_call(
        _tc_stage_body,
        grid=(N_STEPS,),
        in_specs=[
            pl.BlockSpec(memory_space=pltpu.SMEM, block_shape=(1, 1),
                         index_map=lambda s: (0, 0)),
            pl.BlockSpec((T_BLK, D), lambda s: (s + stage * N_STEPS, 0)),
            pl.BlockSpec((T, D), lambda s: (0, 0)),
            pl.BlockSpec((K, D), lambda s: (0, 0)),
        ],
        out_specs=[
            pl.BlockSpec((T_BLK * D,), lambda s: (s,)),
            pl.BlockSpec(memory_space=pltpu.SMEM, block_shape=(1, 1),
                         index_map=lambda s: (0, 0)),
        ],
        out_shape=[
            jax.ShapeDtypeStruct((T_STAGE * D,), jnp.int32),
            jax.ShapeDtypeStruct((1, 1), jnp.float32),
        ],
        scratch_shapes=[
            pltpu.VMEM((K, D), jnp.float32),
            pltpu.SMEM((1, 1), jnp.float32),
        ],
        compiler_params=pltpu.CompilerParams(
            dimension_semantics=("arbitrary",),
        ),
    )(loss_in, x2d, x2d, emb)


_NC = 2
_NS = 16
_NW = _NC * _NS
_B = T * D
_B_STAGE = T_STAGE * D
_ROWS_PER_W = _B_STAGE // _NW
_CHUNK = 128
_N_CHUNKS = _ROWS_PER_W // _CHUNK


def _sc_stage_body(stage, emb_hbm, idx_hbm, out_hbm,
                   idx_all, rows_v0, rows_v1, sem0, sem1):
    wid = lax.axis_index("s") * _NC + lax.axis_index("c")
    w_base = wid * _ROWS_PER_W
    out_base = stage * _B_STAGE + w_base

    pltpu.sync_copy(idx_hbm.at[pl.ds(w_base, _ROWS_PER_W)], idx_all)

    def fire(rows_v, sem, c):
        idx_c = idx_all.at[pl.ds(c * _CHUNK, _CHUNK)]
        pltpu.make_async_copy(emb_hbm.at[idx_c], rows_v, sem).start()

    def drain(rows_v, sem, c):
        idx_c = idx_all.at[pl.ds(c * _CHUNK, _CHUNK)]
        pltpu.make_async_copy(emb_hbm.at[idx_c], rows_v, sem).wait()
        pltpu.sync_copy(rows_v, out_hbm.at[pl.ds(out_base + c * _CHUNK,
                                                 _CHUNK)])

    fire(rows_v0, sem0, 0)
    for c in range(_N_CHUNKS):
        if c + 1 < _N_CHUNKS:
            fire((rows_v1, rows_v0)[c % 2], (sem1, sem0)[c % 2], c + 1)
        drain((rows_v0, rows_v1)[c % 2], (sem0, sem1)[c % 2], c)


@functools.cache
def _sc_stage_kernel(stage):
    return pl.kernel(
        functools.partial(_sc_stage_body, stage),
        out_type=(),
        mesh=plsc.VectorSubcoreMesh(core_axis_name="c", subcore_axis_name="s",
                                    num_cores=_NC, num_subcores=_NS),
        scratch_types=[
            pltpu.VMEM((_ROWS_PER_W,), jnp.int32),
            pltpu.VMEM((_CHUNK, D), jnp.float32),
            pltpu.VMEM((_CHUNK, D), jnp.float32),
            pltpu.SemaphoreType.DMA,
            pltpu.SemaphoreType.DMA,
        ],
        name=f"sc_gather_stage{stage}",
    )


def kernel(x, embeddings):
    x2d = x[0]
    out_ref = jax.new_ref(pl.empty((_B, D), jnp.float32))
    loss = jnp.zeros((1, 1), jnp.float32)
    for s in range(N_STAGES):
        idx_s, loss = _tc_stage_call(s, x2d, embeddings, loss)
        _sc_stage_kernel(s)(embeddings, idx_s, out_ref)
    return out_ref[...].reshape(1, T, D, D), loss[0, 0]


# speedup vs baseline: 1.0747x; 1.0057x over previous
import functools

import jax
import jax.numpy as jnp
from jax import lax
from jax.experimental import pallas as pl
from jax.experimental.pallas import tpu as pltpu
from jax.experimental.pallas import tpu_sc as plsc

K = 512
D = 256
T = 256
COMMITMENT = 0.25

N_STAGES = 4
T_STAGE = T // N_STAGES
T_BLK = 16
N_STEPS = T_STAGE // T_BLK
KC = 128

_LOSS_SCALE = (1.0 + COMMITMENT) / (T * D * D)


def _tc_stage_body(loss_in_ref, x_blk_ref, x_full_ref, emb_ref,
                   idx_ref, loss_ref, f_ref, acc_ref):
    s = pl.program_id(0)
    emb = emb_ref[...]

    @pl.when(s == 0)
    def _init():
        x_full = x_full_ref[...]
        g = lax.dot_general(emb, x_full, (((1,), (1,)), ((), ())),
                            preferred_element_type=jnp.float32)
        enorm2 = jnp.sum(emb * emb, axis=1, keepdims=True)
        f_ref[...] = enorm2 - 2.0 * g
        acc_ref[0, 0] = (loss_in_ref[0, 0]
                         + _LOSS_SCALE * float(T_STAGE)
                         * jnp.sum(x_full * x_full))

    xblk = x_blk_ref[...]

    f = f_ref[...]
    m = None
    idx = None
    lvals = None
    for c in range(K // KC):
        e_c = emb[c * KC:(c + 1) * KC, :]
        f_c = f[c * KC:(c + 1) * KC, :]
        diff = xblk[:, None, :] - e_c[None, :, :]
        d = diff * diff
        mc = jnp.min(d, axis=1)
        iota_c = lax.broadcasted_iota(jnp.int32, (T_BLK, KC, D), 1) + c * KC
        hit_c = d == mc[:, None, :]
        idxc = jnp.min(jnp.where(hit_c, iota_c, K), axis=1)
        onehot_c = iota_c == idxc[:, None, :]
        lc = jnp.sum(jnp.where(onehot_c, f_c[None, :, :], 0.0), axis=1)
        if m is None:
            m, idx, lvals = mc, idxc, lc
        else:
            upd = mc < m
            idx = jnp.where(upd, idxc, idx)
            lvals = jnp.where(upd, lc, lvals)
            m = jnp.where(upd, mc, m)
    idx_ref[...] = idx.reshape(T_BLK * D)
    acc_ref[0, 0] += _LOSS_SCALE * jnp.sum(lvals)

    @pl.when(s == N_STEPS - 1)
    def _fin():
        loss_ref[0, 0] = acc_ref[0, 0]


def _tc_stage_call(stage, x2d, emb, loss_in):
    return pl.pallas_call(
        _tc_stage_body,
        grid=(N_STEPS,),
        in_specs=[
            pl.BlockSpec(memory_space=pltpu.SMEM, block_shape=(1, 1),
                         index_map=lambda s: (0, 0)),
            pl.BlockSpec((T_BLK, D), lambda s: (s + stage * N_STEPS, 0)),
            pl.BlockSpec((T, D), lambda s: (0, 0)),
            pl.BlockSpec((K, D), lambda s: (0, 0)),
        ],
        out_specs=[
            pl.BlockSpec((T_BLK * D,), lambda s: (s,)),
            pl.BlockSpec(memory_space=pltpu.SMEM, block_shape=(1, 1),
                         index_map=lambda s: (0, 0)),
        ],
        out_shape=[
            jax.ShapeDtypeStruct((T_STAGE * D,), jnp.int32),
            jax.ShapeDtypeStruct((1, 1), jnp.float32),
        ],
        scratch_shapes=[
            pltpu.VMEM((K, D), jnp.float32),
            pltpu.SMEM((1, 1), jnp.float32),
        ],
        compiler_params=pltpu.CompilerParams(
            dimension_semantics=("arbitrary",),
        ),
    )(loss_in, x2d, x2d, emb)


_NC = 2
_NS = 16
_NW = _NC * _NS
_B = T * D
_B_STAGE = T_STAGE * D
_ROWS_PER_W = _B_STAGE // _NW
_CHUNK = 128
_N_CHUNKS = _ROWS_PER_W // _CHUNK


def _sc_stage_body(stage, emb_hbm, idx_hbm, out_hbm,
                   idx_all, rows_v0, rows_v1, sem0, sem1):
    wid = lax.axis_index("s") * _NC + lax.axis_index("c")
    w_base = wid * _ROWS_PER_W
    out_base = stage * _B_STAGE + w_base

    pltpu.sync_copy(idx_hbm.at[pl.ds(w_base, _ROWS_PER_W)], idx_all)

    def fire(rows_v, sem, c):
        idx_c = idx_all.at[pl.ds(c * _CHUNK, _CHUNK)]
        pltpu.make_async_copy(emb_hbm.at[idx_c], rows_v, sem).start()

    def drain(rows_v, sem, c):
        idx_c = idx_all.at[pl.ds(c * _CHUNK, _CHUNK)]
        pltpu.make_async_copy(emb_hbm.at[idx_c], rows_v, sem).wait()
        pltpu.sync_copy(rows_v, out_hbm.at[pl.ds(out_base + c * _CHUNK,
                                                 _CHUNK)])

    fire(rows_v0, sem0, 0)
    for c in range(_N_CHUNKS):
        if c + 1 < _N_CHUNKS:
            fire((rows_v1, rows_v0)[c % 2], (sem1, sem0)[c % 2], c + 1)
        drain((rows_v0, rows_v1)[c % 2], (sem0, sem1)[c % 2], c)


@functools.cache
def _sc_stage_kernel(stage):
    return pl.kernel(
        functools.partial(_sc_stage_body, stage),
        out_type=(),
        mesh=plsc.VectorSubcoreMesh(core_axis_name="c", subcore_axis_name="s",
                                    num_cores=_NC, num_subcores=_NS),
        scratch_types=[
            pltpu.VMEM((_ROWS_PER_W,), jnp.int32),
            pltpu.VMEM((_CHUNK, D), jnp.float32),
            pltpu.VMEM((_CHUNK, D), jnp.float32),
            pltpu.SemaphoreType.DMA,
            pltpu.SemaphoreType.DMA,
        ],
        name=f"sc_gather_stage{stage}",
    )


def kernel(x, embeddings):
    x2d = x[0]
    out_ref = jax.new_ref(pl.empty((_B, D), jnp.float32))
    loss = jnp.zeros((1, 1), jnp.float32)
    for s in range(N_STAGES):
        idx_s, loss = _tc_stage_call(s, x2d, embeddings, loss)
        _sc_stage_kernel(s)(embeddings, idx_s, out_ref)
    return out_ref[...].reshape(1, T, D, D), loss[0, 0]
